# Initial kernel scaffold; baseline (speedup 1.0000x reference)
import jax
import jax.numpy as jnp
from jax.experimental import pallas as pl


def kernel(x, edge_index, W1, b1, W2, b2):
    raise NotImplementedError("write your pallas kernel here")


# speedup vs baseline: 31.9027x; 31.9027x over previous
import functools

import jax
import jax.numpy as jnp
from jax import lax
from jax.experimental import pallas as pl
from jax.experimental.pallas import tpu as pltpu
from jax.experimental.pallas import tpu_sc as plsc

N = 10000
NP = 10240
F = 16
NC = 2
NS = 16
NW = NC * NS
C = 128
SLICE = NP // NS
BLK = 1280
GRID = NP // BLK

_mesh = plsc.VectorSubcoreMesh(
    core_axis_name="c", subcore_axis_name="s", num_cores=NC, num_subcores=NS)
_sc_params = pltpu.CompilerParams(use_tc_tiling_on_sc=False)


def _zero_fill(ref, n_rows):
    def body(j, _):
        ref[j, :] = jnp.zeros((16,), jnp.float32)
        return 0
    lax.fori_loop(0, n_rows, body, 0)


def _make_sc_degree(K):
    @functools.partial(
        pl.kernel,
        out_type=jax.ShapeDtypeStruct((NC, NP, F), jnp.float32),
        mesh=_mesh,
        compiler_params=_sc_params,
        scratch_types=[
            pltpu.VMEM((K, C), jnp.int32),
            pltpu.VMEM((C, F), jnp.float32),
            pltpu.VMEM((SLICE, F), jnp.float32),
            pltpu.VMEM_SHARED((NP, F), jnp.float32),
        ],
    )
    def deg_kernel(dst_hbm, out_hbm, dstv, onesv, zerov, acc):
        c = lax.axis_index("c")
        s = lax.axis_index("s")
        t = c * NS + s

        def fill_ones(j, _):
            onesv[j, :] = jnp.ones((16,), jnp.float32)
            return 0
        lax.fori_loop(0, C, fill_ones, 0)
        _zero_fill(zerov, SLICE)
        pltpu.sync_copy(zerov, acc.at[pl.ds(s * SLICE, SLICE)])
        pltpu.sync_copy(dst_hbm.at[t], dstv)
        plsc.subcore_barrier()

        def step(j, _):
            pltpu.sync_copy(onesv, acc.at[dstv.at[j]], add=True)
            return 0
        lax.fori_loop(0, K, step, 0)
        plsc.subcore_barrier()
        pltpu.sync_copy(acc.at[pl.ds(s * SLICE, SLICE)],
                        out_hbm.at[c, pl.ds(s * SLICE, SLICE)])

    return deg_kernel


def _make_sc_agg(K):
    @functools.partial(
        pl.kernel,
        out_type=jax.ShapeDtypeStruct((NC, NP, F), jnp.float32),
        mesh=_mesh,
        compiler_params=_sc_params,
        scratch_types=[
            pltpu.VMEM((K, C), jnp.int32),
            pltpu.VMEM((K, C), jnp.int32),
            pltpu.VMEM((C, F), jnp.float32),
            pltpu.VMEM((SLICE, F), jnp.float32),
            pltpu.VMEM_SHARED((NP, F), jnp.float32),
            pltpu.SemaphoreType.DMA,
        ],
    )
    def agg_kernel(table_hbm, src_hbm, dst_hbm, out_hbm,
                   srcv, dstv, rows, zerov, acc, gsem):
        c = lax.axis_index("c")
        s = lax.axis_index("s")
        t = c * NS + s

        _zero_fill(zerov, SLICE)
        pltpu.sync_copy(zerov, acc.at[pl.ds(s * SLICE, SLICE)])
        pltpu.sync_copy(src_hbm.at[t], srcv)
        pltpu.sync_copy(dst_hbm.at[t], dstv)
        plsc.subcore_barrier()

        def step(j, _):
            pltpu.async_copy(table_hbm.at[srcv.at[j]], rows, gsem).wait()
            pltpu.sync_copy(rows, acc.at[dstv.at[j]], add=True)
            return 0
        lax.fori_loop(0, K, step, 0)
        plsc.subcore_barrier()
        pltpu.sync_copy(acc.at[pl.ds(s * SLICE, SLICE)],
                        out_hbm.at[c, pl.ds(s * SLICE, SLICE)])

    return agg_kernel


def _tc_scale1(x_p, W1, deg_part):
    def body(xr, w1r, dpr, h1sr, dinvr):
        deg = dpr[0, :, 0:1] + dpr[1, :, 0:1] + 1.0
        dinv = lax.rsqrt(deg)
        h = jnp.dot(xr[...], w1r[...], preferred_element_type=jnp.float32)
        h1sr[...] = h * dinv
        dinvr[...] = jnp.broadcast_to(dinv, (BLK, F))

    return pl.pallas_call(
        body,
        grid=(GRID,),
        in_specs=[
            pl.BlockSpec((BLK, 128), lambda i: (i, 0)),
            pl.BlockSpec((128, F), lambda i: (0, 0)),
            pl.BlockSpec((NC, BLK, F), lambda i: (0, i, 0)),
        ],
        out_specs=[
            pl.BlockSpec((BLK, F), lambda i: (i, 0)),
            pl.BlockSpec((BLK, F), lambda i: (i, 0)),
        ],
        out_shape=[
            jax.ShapeDtypeStruct((NP, F), jnp.float32),
            jax.ShapeDtypeStruct((NP, F), jnp.float32),
        ],
    )(x_p, W1, deg_part)


def _tc_mid(agg1, h1s, dinv, W2p, b1r):
    def body(ar, h1r, dr, w2r, b1ref, outr):
        i = pl.program_id(0)
        agg = ar[0] + ar[1]
        o1 = jnp.maximum(dr[...] * (agg + h1r[...]) + b1ref[...], 0.0)
        h2 = jnp.dot(o1, w2r[...], preferred_element_type=jnp.float32)
        h2s = h2 * dr[...]
        row = i * BLK + lax.broadcasted_iota(jnp.int32, (BLK, F), 0)
        outr[...] = jnp.where(row < N, h2s, 0.0)

    return pl.pallas_call(
        body,
        grid=(GRID,),
        in_specs=[
            pl.BlockSpec((NC, BLK, F), lambda i: (0, i, 0)),
            pl.BlockSpec((BLK, F), lambda i: (i, 0)),
            pl.BlockSpec((BLK, F), lambda i: (i, 0)),
            pl.BlockSpec((F, F), lambda i: (0, 0)),
            pl.BlockSpec((1, F), lambda i: (0, 0)),
        ],
        out_specs=pl.BlockSpec((BLK, F), lambda i: (i, 0)),
        out_shape=jax.ShapeDtypeStruct((NP, F), jnp.float32),
    )(agg1, h1s, dinv, W2p, b1r)


def _tc_final(agg2, h2s, dinv, b2r):
    def body(ar, h2r, dr, b2ref, outr):
        outr[...] = dr[...] * (ar[0] + ar[1] + h2r[...]) + b2ref[...]

    return pl.pallas_call(
        body,
        grid=(GRID,),
        in_specs=[
            pl.BlockSpec((NC, BLK, F), lambda i: (0, i, 0)),
            pl.BlockSpec((BLK, F), lambda i: (i, 0)),
            pl.BlockSpec((BLK, F), lambda i: (i, 0)),
            pl.BlockSpec((1, F), lambda i: (0, 0)),
        ],
        out_specs=pl.BlockSpec((BLK, F), lambda i: (i, 0)),
        out_shape=jax.ShapeDtypeStruct((NP, F), jnp.float32),
    )(agg2, h2s, dinv, b2r)


def kernel(x, edge_index, W1, b1, W2, b2):
    E = edge_index.shape[1]
    K = -(-E // (NW * C))
    EPAD = NW * K * C

    src = edge_index[0].astype(jnp.int32)
    dst = edge_index[1].astype(jnp.int32)
    pad = EPAD - E
    src3 = jnp.concatenate(
        [src, jnp.full((pad,), N, jnp.int32)]).reshape(NW, K, C)
    dst3 = jnp.concatenate(
        [dst, jnp.full((pad,), N, jnp.int32)]).reshape(NW, K, C)
    x_p = jnp.pad(x, ((0, NP - N), (0, 0)))
    W2p = jnp.pad(W2, ((0, 0), (0, F - W2.shape[1])))
    b1r = b1.reshape(1, F)
    b2r = jnp.pad(b2, (0, F - b2.shape[0])).reshape(1, F)

    sc_degree = _make_sc_degree(K)
    sc_agg = _make_sc_agg(K)

    deg_part = sc_degree(dst3)
    h1s, dinv = _tc_scale1(x_p, W1, deg_part)
    agg1 = sc_agg(h1s, src3, dst3)
    h2s = _tc_mid(agg1, h1s, dinv, W2p, b1r)
    agg2 = sc_agg(h2s, src3, dst3)
    outf = _tc_final(agg2, h2s, dinv, b2r)
    return outf[:N, :W2.shape[1]]


# speedup vs baseline: 35.4377x; 1.1108x over previous
import functools

import jax
import jax.numpy as jnp
from jax import lax
from jax.experimental import pallas as pl
from jax.experimental.pallas import tpu as pltpu
from jax.experimental.pallas import tpu_sc as plsc

N = 10000
NP = 10240
F = 16
NC = 2
NS = 16
NW = NC * NS
C = 128
SLICE = NP // NS
BLK = 1280
GRID = NP // BLK

_mesh = plsc.VectorSubcoreMesh(
    core_axis_name="c", subcore_axis_name="s", num_cores=NC, num_subcores=NS)
_sc_params = pltpu.CompilerParams(use_tc_tiling_on_sc=False)


def _zero_fill(ref, n_rows):
    def body(j, _):
        ref[j, :] = jnp.zeros((16,), jnp.float32)
        return 0
    lax.fori_loop(0, n_rows, body, 0)


def _make_sc_degree(K):
    @functools.partial(
        pl.kernel,
        out_type=jax.ShapeDtypeStruct((NC, NP, F), jnp.float32),
        mesh=_mesh,
        compiler_params=_sc_params,
        scratch_types=[
            pltpu.VMEM((K, C), jnp.int32),
            pltpu.VMEM((C, F), jnp.float32),
            pltpu.VMEM((SLICE, F), jnp.float32),
            pltpu.VMEM_SHARED((NP, F), jnp.float32),
            pltpu.SemaphoreType.DMA,
        ],
    )
    def deg_kernel(dst_hbm, out_hbm, dstv, onesv, zerov, acc, dsem):
        c = lax.axis_index("c")
        s = lax.axis_index("s")
        t = c * NS + s

        def fill_ones(j, _):
            onesv[j, :] = jnp.ones((16,), jnp.float32)
            return 0
        lax.fori_loop(0, C, fill_ones, 0)
        _zero_fill(zerov, SLICE)
        pltpu.sync_copy(zerov, acc.at[pl.ds(s * SLICE, SLICE)])
        pltpu.sync_copy(dst_hbm.at[t], dstv)
        plsc.subcore_barrier()

        def step(jj, _):
            base = 8 * jj
            for u in range(8):
                pltpu.async_copy(onesv, acc.at[dstv.at[base + u]], dsem,
                                 add=True)
            for u in range(8):
                pltpu.make_async_copy(onesv, acc.at[dstv.at[base + u]],
                                      dsem).wait()
            return 0
        lax.fori_loop(0, K // 8, step, 0)

        def step_tail(j, _):
            pltpu.sync_copy(onesv, acc.at[dstv.at[j]], add=True)
            return 0
        lax.fori_loop((K // 8) * 8, K, step_tail, 0)
        plsc.subcore_barrier()
        pltpu.sync_copy(acc.at[pl.ds(s * SLICE, SLICE)],
                        out_hbm.at[c, pl.ds(s * SLICE, SLICE)])

    return deg_kernel


def _make_sc_agg(K):
    @functools.partial(
        pl.kernel,
        out_type=jax.ShapeDtypeStruct((NC, NP, F), jnp.float32),
        mesh=_mesh,
        compiler_params=_sc_params,
        scratch_types=[
            pltpu.VMEM((K, C), jnp.int32),
            pltpu.VMEM((K, C), jnp.int32),
            pltpu.VMEM((C, F), jnp.float32),
            pltpu.VMEM((C, F), jnp.float32),
            pltpu.VMEM((SLICE, F), jnp.float32),
            pltpu.VMEM_SHARED((NP, F), jnp.float32),
            pltpu.SemaphoreType.DMA,
            pltpu.SemaphoreType.DMA,
        ],
    )
    def agg_kernel(table_hbm, src_hbm, dst_hbm, out_hbm,
                   srcv, dstv, rows0, rows1, zerov, acc, sem0, sem1):
        c = lax.axis_index("c")
        s = lax.axis_index("s")
        t = c * NS + s

        _zero_fill(zerov, SLICE)
        pltpu.sync_copy(zerov, acc.at[pl.ds(s * SLICE, SLICE)])
        pltpu.sync_copy(src_hbm.at[t], srcv)
        pltpu.sync_copy(dst_hbm.at[t], dstv)
        plsc.subcore_barrier()

        pltpu.async_copy(table_hbm.at[srcv.at[0]], rows0, sem0)

        def step(jj, _):
            j0 = 2 * jj
            pltpu.async_copy(table_hbm.at[srcv.at[j0 + 1]], rows1, sem1)
            pltpu.make_async_copy(table_hbm.at[srcv.at[j0]], rows0,
                                  sem0).wait()
            pltpu.sync_copy(rows0, acc.at[dstv.at[j0]], add=True)
            jn = jnp.minimum(j0 + 2, K - 1)
            pltpu.async_copy(table_hbm.at[srcv.at[jn]], rows0, sem0)
            pltpu.make_async_copy(table_hbm.at[srcv.at[j0 + 1]], rows1,
                                  sem1).wait()
            pltpu.sync_copy(rows1, acc.at[dstv.at[j0 + 1]], add=True)
            return 0
        lax.fori_loop(0, K // 2, step, 0)
        pltpu.make_async_copy(table_hbm.at[srcv.at[K - 1]], rows0,
                              sem0).wait()
        plsc.subcore_barrier()
        pltpu.sync_copy(acc.at[pl.ds(s * SLICE, SLICE)],
                        out_hbm.at[c, pl.ds(s * SLICE, SLICE)])

    return agg_kernel


def _tc_scale1(x_p, W1, deg_part):
    def body(xr, w1r, dpr, h1sr, dinvr):
        deg = dpr[0, :, 0:1] + dpr[1, :, 0:1] + 1.0
        dinv = lax.rsqrt(deg)
        h = jnp.dot(xr[...], w1r[...], preferred_element_type=jnp.float32)
        h1sr[...] = h * dinv
        dinvr[...] = jnp.broadcast_to(dinv, (BLK, F))

    return pl.pallas_call(
        body,
        grid=(GRID,),
        in_specs=[
            pl.BlockSpec((BLK, 128), lambda i: (i, 0)),
            pl.BlockSpec((128, F), lambda i: (0, 0)),
            pl.BlockSpec((NC, BLK, F), lambda i: (0, i, 0)),
        ],
        out_specs=[
            pl.BlockSpec((BLK, F), lambda i: (i, 0)),
            pl.BlockSpec((BLK, F), lambda i: (i, 0)),
        ],
        out_shape=[
            jax.ShapeDtypeStruct((NP, F), jnp.float32),
            jax.ShapeDtypeStruct((NP, F), jnp.float32),
        ],
    )(x_p, W1, deg_part)


def _tc_mid(agg1, h1s, dinv, W2p, b1r):
    def body(ar, h1r, dr, w2r, b1ref, outr):
        i = pl.program_id(0)
        agg = ar[0] + ar[1]
        o1 = jnp.maximum(dr[...] * (agg + h1r[...]) + b1ref[...], 0.0)
        h2 = jnp.dot(o1, w2r[...], preferred_element_type=jnp.float32)
        h2s = h2 * dr[...]
        row = i * BLK + lax.broadcasted_iota(jnp.int32, (BLK, F), 0)
        outr[...] = jnp.where(row < N, h2s, 0.0)

    return pl.pallas_call(
        body,
        grid=(GRID,),
        in_specs=[
            pl.BlockSpec((NC, BLK, F), lambda i: (0, i, 0)),
            pl.BlockSpec((BLK, F), lambda i: (i, 0)),
            pl.BlockSpec((BLK, F), lambda i: (i, 0)),
            pl.BlockSpec((F, F), lambda i: (0, 0)),
            pl.BlockSpec((1, F), lambda i: (0, 0)),
        ],
        out_specs=pl.BlockSpec((BLK, F), lambda i: (i, 0)),
        out_shape=jax.ShapeDtypeStruct((NP, F), jnp.float32),
    )(agg1, h1s, dinv, W2p, b1r)


def _tc_final(agg2, h2s, dinv, b2r):
    def body(ar, h2r, dr, b2ref, outr):
        outr[...] = dr[...] * (ar[0] + ar[1] + h2r[...]) + b2ref[...]

    return pl.pallas_call(
        body,
        grid=(GRID,),
        in_specs=[
            pl.BlockSpec((NC, BLK, F), lambda i: (0, i, 0)),
            pl.BlockSpec((BLK, F), lambda i: (i, 0)),
            pl.BlockSpec((BLK, F), lambda i: (i, 0)),
            pl.BlockSpec((1, F), lambda i: (0, 0)),
        ],
        out_specs=pl.BlockSpec((BLK, F), lambda i: (i, 0)),
        out_shape=jax.ShapeDtypeStruct((NP, F), jnp.float32),
    )(agg2, h2s, dinv, b2r)


def kernel(x, edge_index, W1, b1, W2, b2):
    E = edge_index.shape[1]
    K = -(-E // (NW * C))
    K += K % 2
    EPAD = NW * K * C

    src = edge_index[0].astype(jnp.int32)
    dst = edge_index[1].astype(jnp.int32)
    pad = EPAD - E
    src3 = jnp.concatenate(
        [src, jnp.full((pad,), N, jnp.int32)]).reshape(NW, K, C)
    dst3 = jnp.concatenate(
        [dst, jnp.full((pad,), N, jnp.int32)]).reshape(NW, K, C)
    x_p = jnp.pad(x, ((0, NP - N), (0, 0)))
    W2p = jnp.pad(W2, ((0, 0), (0, F - W2.shape[1])))
    b1r = b1.reshape(1, F)
    b2r = jnp.pad(b2, (0, F - b2.shape[0])).reshape(1, F)

    sc_degree = _make_sc_degree(K)
    sc_agg = _make_sc_agg(K)

    deg_part = sc_degree(dst3)
    h1s, dinv = _tc_scale1(x_p, W1, deg_part)
    agg1 = sc_agg(h1s, src3, dst3)
    h2s = _tc_mid(agg1, h1s, dinv, W2p, b1r)
    agg2 = sc_agg(h2s, src3, dst3)
    outf = _tc_final(agg2, h2s, dinv, b2r)
    return outf[:N, :W2.shape[1]]


# speedup vs baseline: 35.5399x; 1.0029x over previous
import functools

import jax
import jax.numpy as jnp
from jax import lax
from jax.experimental import pallas as pl
from jax.experimental.pallas import tpu as pltpu
from jax.experimental.pallas import tpu_sc as plsc

N = 10000
NP = 10240
F = 16
NC = 2
NS = 16
NW = NC * NS
C = 128
NBUF = 4
SLICE = NP // NS
BLK = 1280
GRID = NP // BLK

_mesh = plsc.VectorSubcoreMesh(
    core_axis_name="c", subcore_axis_name="s", num_cores=NC, num_subcores=NS)
_sc_params = pltpu.CompilerParams(use_tc_tiling_on_sc=False)


def _zero_fill(ref, n_rows):
    def body(j, _):
        ref[j, :] = jnp.zeros((16,), jnp.float32)
        return 0
    lax.fori_loop(0, n_rows, body, 0)


def _make_sc_degree(K):
    @functools.partial(
        pl.kernel,
        out_type=jax.ShapeDtypeStruct((NC, NP, F), jnp.float32),
        mesh=_mesh,
        compiler_params=_sc_params,
        scratch_types=[
            pltpu.VMEM((K, C), jnp.int32),
            pltpu.VMEM((C, F), jnp.float32),
            pltpu.VMEM((SLICE, F), jnp.float32),
            pltpu.VMEM_SHARED((NP, F), jnp.float32),
            pltpu.SemaphoreType.DMA,
        ],
    )
    def deg_kernel(dst_hbm, out_hbm, dstv, onesv, zerov, acc, dsem):
        c = lax.axis_index("c")
        s = lax.axis_index("s")
        t = c * NS + s

        def fill_ones(j, _):
            onesv[j, :] = jnp.ones((16,), jnp.float32)
            return 0
        lax.fori_loop(0, C, fill_ones, 0)
        _zero_fill(zerov, SLICE)
        pltpu.sync_copy(zerov, acc.at[pl.ds(s * SLICE, SLICE)])
        pltpu.sync_copy(dst_hbm.at[t], dstv)
        plsc.subcore_barrier()

        def step(jj, _):
            base = 8 * jj
            for u in range(8):
                pltpu.async_copy(onesv, acc.at[dstv.at[base + u]], dsem,
                                 add=True)
            for u in range(8):
                pltpu.make_async_copy(onesv, acc.at[dstv.at[base + u]],
                                      dsem).wait()
            return 0
        lax.fori_loop(0, K // 8, step, 0)

        def step_tail(j, _):
            pltpu.sync_copy(onesv, acc.at[dstv.at[j]], add=True)
            return 0
        lax.fori_loop((K // 8) * 8, K, step_tail, 0)
        plsc.subcore_barrier()
        pltpu.sync_copy(acc.at[pl.ds(s * SLICE, SLICE)],
                        out_hbm.at[c, pl.ds(s * SLICE, SLICE)])

    return deg_kernel


def _make_sc_agg(K):
    @functools.partial(
        pl.kernel,
        out_type=jax.ShapeDtypeStruct((NC, NP, F), jnp.float32),
        mesh=_mesh,
        compiler_params=_sc_params,
        scratch_types=[
            pltpu.VMEM((K, C), jnp.int32),
            pltpu.VMEM((K, C), jnp.int32),
            [pltpu.VMEM((C, F), jnp.float32) for _ in range(NBUF)],
            pltpu.VMEM((SLICE, F), jnp.float32),
            pltpu.VMEM_SHARED((NP, F), jnp.float32),
            [pltpu.SemaphoreType.DMA for _ in range(NBUF)],
        ],
    )
    def agg_kernel(table_hbm, src_hbm, dst_hbm, out_hbm,
                   srcv, dstv, rows, zerov, acc, sems):
        c = lax.axis_index("c")
        s = lax.axis_index("s")
        t = c * NS + s

        _zero_fill(zerov, SLICE)
        pltpu.sync_copy(zerov, acc.at[pl.ds(s * SLICE, SLICE)])
        pltpu.sync_copy(src_hbm.at[t], srcv)
        pltpu.sync_copy(dst_hbm.at[t], dstv)
        plsc.subcore_barrier()

        for u in range(NBUF):
            pltpu.async_copy(table_hbm.at[srcv.at[u]], rows[u], sems[u])

        def step(jj, _):
            base = NBUF * jj
            for u in range(NBUF):
                j = base + u
                pltpu.make_async_copy(table_hbm.at[srcv.at[j]], rows[u],
                                      sems[u]).wait()
                pltpu.sync_copy(rows[u], acc.at[dstv.at[j]], add=True)
                jn = jnp.minimum(j + NBUF, K - 1)
                pltpu.async_copy(table_hbm.at[srcv.at[jn]], rows[u], sems[u])
            return 0
        lax.fori_loop(0, K // NBUF, step, 0)
        for u in range(NBUF):
            pltpu.make_async_copy(table_hbm.at[srcv.at[K - 1]], rows[u],
                                  sems[u]).wait()
        plsc.subcore_barrier()
        pltpu.sync_copy(acc.at[pl.ds(s * SLICE, SLICE)],
                        out_hbm.at[c, pl.ds(s * SLICE, SLICE)])

    return agg_kernel


def _tc_scale1(x_p, W1, deg_part):
    def body(xr, w1r, dpr, h1sr, dinvr):
        deg = dpr[0, :, 0:1] + dpr[1, :, 0:1] + 1.0
        dinv = lax.rsqrt(deg)
        h = jnp.dot(xr[...], w1r[...], preferred_element_type=jnp.float32)
        h1sr[...] = h * dinv
        dinvr[...] = jnp.broadcast_to(dinv, (BLK, F))

    return pl.pallas_call(
        body,
        grid=(GRID,),
        in_specs=[
            pl.BlockSpec((BLK, 128), lambda i: (i, 0)),
            pl.BlockSpec((128, F), lambda i: (0, 0)),
            pl.BlockSpec((NC, BLK, F), lambda i: (0, i, 0)),
        ],
        out_specs=[
            pl.BlockSpec((BLK, F), lambda i: (i, 0)),
            pl.BlockSpec((BLK, F), lambda i: (i, 0)),
        ],
        out_shape=[
            jax.ShapeDtypeStruct((NP, F), jnp.float32),
            jax.ShapeDtypeStruct((NP, F), jnp.float32),
        ],
    )(x_p, W1, deg_part)


def _tc_mid(agg1, h1s, dinv, W2p, b1r):
    def body(ar, h1r, dr, w2r, b1ref, outr):
        i = pl.program_id(0)
        agg = ar[0] + ar[1]
        o1 = jnp.maximum(dr[...] * (agg + h1r[...]) + b1ref[...], 0.0)
        h2 = jnp.dot(o1, w2r[...], preferred_element_type=jnp.float32)
        h2s = h2 * dr[...]
        row = i * BLK + lax.broadcasted_iota(jnp.int32, (BLK, F), 0)
        outr[...] = jnp.where(row < N, h2s, 0.0)

    return pl.pallas_call(
        body,
        grid=(GRID,),
        in_specs=[
            pl.BlockSpec((NC, BLK, F), lambda i: (0, i, 0)),
            pl.BlockSpec((BLK, F), lambda i: (i, 0)),
            pl.BlockSpec((BLK, F), lambda i: (i, 0)),
            pl.BlockSpec((F, F), lambda i: (0, 0)),
            pl.BlockSpec((1, F), lambda i: (0, 0)),
        ],
        out_specs=pl.BlockSpec((BLK, F), lambda i: (i, 0)),
        out_shape=jax.ShapeDtypeStruct((NP, F), jnp.float32),
    )(agg1, h1s, dinv, W2p, b1r)


def _tc_final(agg2, h2s, dinv, b2r):
    def body(ar, h2r, dr, b2ref, outr):
        outr[...] = dr[...] * (ar[0] + ar[1] + h2r[...]) + b2ref[...]

    return pl.pallas_call(
        body,
        grid=(GRID,),
        in_specs=[
            pl.BlockSpec((NC, BLK, F), lambda i: (0, i, 0)),
            pl.BlockSpec((BLK, F), lambda i: (i, 0)),
            pl.BlockSpec((BLK, F), lambda i: (i, 0)),
            pl.BlockSpec((1, F), lambda i: (0, 0)),
        ],
        out_specs=pl.BlockSpec((BLK, F), lambda i: (i, 0)),
        out_shape=jax.ShapeDtypeStruct((NP, F), jnp.float32),
    )(agg2, h2s, dinv, b2r)


def kernel(x, edge_index, W1, b1, W2, b2):
    E = edge_index.shape[1]
    K = -(-E // (NW * C))
    K += (-K) % NBUF
    EPAD = NW * K * C

    src = edge_index[0].astype(jnp.int32)
    dst = edge_index[1].astype(jnp.int32)
    pad = EPAD - E
    src3 = jnp.concatenate(
        [src, jnp.full((pad,), N, jnp.int32)]).reshape(NW, K, C)
    dst3 = jnp.concatenate(
        [dst, jnp.full((pad,), N, jnp.int32)]).reshape(NW, K, C)
    x_p = jnp.pad(x, ((0, NP - N), (0, 0)))
    W2p = jnp.pad(W2, ((0, 0), (0, F - W2.shape[1])))
    b1r = b1.reshape(1, F)
    b2r = jnp.pad(b2, (0, F - b2.shape[0])).reshape(1, F)

    sc_degree = _make_sc_degree(K)
    sc_agg = _make_sc_agg(K)

    deg_part = sc_degree(dst3)
    h1s, dinv = _tc_scale1(x_p, W1, deg_part)
    agg1 = sc_agg(h1s, src3, dst3)
    h2s = _tc_mid(agg1, h1s, dinv, W2p, b1r)
    agg2 = sc_agg(h2s, src3, dst3)
    outf = _tc_final(agg2, h2s, dinv, b2r)
    return outf[:N, :W2.shape[1]]
